# baseline (speedup 1.0000x reference)
import functools

import jax
import jax.numpy as jnp
from jax import lax
from jax.experimental import pallas as pl
from jax.experimental.pallas import tpu as pltpu
from jax.experimental.pallas import tpu_sc as plsc

_B = 16384
_N = 11
_D = 32
_NC = 2
_NS = 16
_NW = _NC * _NS
_PER_W = _B // _NW
_CH = 128
_NCH = _PER_W // _CH
_LANES = 16


def _recsys_call(sess_flat, prods_flat, uembT, pemb):
    mesh = plsc.VectorSubcoreMesh(
        core_axis_name="c", subcore_axis_name="s",
        num_cores=_NC, num_subcores=_NS)

    @functools.partial(
        pl.kernel,
        out_type=jax.ShapeDtypeStruct((_B, _N), jnp.float32),
        mesh=mesh,
        compiler_params=pltpu.CompilerParams(
            use_tc_tiling_on_sc=False, needs_layout_passes=False),
        scratch_types=[
            pltpu.VMEM((_CH,), jnp.int32),
            pltpu.VMEM((_N * _CH,), jnp.int32),
            pltpu.VMEM((_D, _CH), jnp.float32),
            pltpu.VMEM((_CH * _N, _D), jnp.float32),
            pltpu.VMEM((_CH, _N), jnp.float32),
            pltpu.SemaphoreType.DMA,
            pltpu.SemaphoreType.DMA,
        ],
    )
    def body(sess_hbm, prods_hbm, uembT_hbm, pemb_hbm, out_hbm,
             sidx, pidx, ubufT, prows, outv, usem, psem):
        wid = lax.axis_index("c") * _NS + lax.axis_index("s")

        def chunk_body(c, carry):
            gbase = wid * _PER_W + c * _CH
            pltpu.sync_copy(sess_hbm.at[pl.ds(gbase, _CH)], sidx)
            pltpu.sync_copy(
                prods_hbm.at[pl.ds(gbase * _N, _N * _CH)], pidx)

            ucps = []
            for d in range(_D):
                cp = pltpu.make_async_copy(
                    uembT_hbm.at[d].at[sidx], ubufT.at[d], usem)
                cp.start()
                ucps.append(cp)
            pcps = []
            for j in range(_N):
                cp = pltpu.make_async_copy(
                    pemb_hbm.at[pidx.at[pl.ds(j * _CH, _CH)]],
                    prows.at[pl.ds(j * _CH, _CH)], psem)
                cp.start()
                pcps.append(cp)
            for cp in ucps:
                cp.wait()
            for cp in pcps:
                cp.wait()

            def group_body(g, carry2):
                bvec = g * _LANES + lax.iota(jnp.int32, _LANES)
                us = [ubufT[d, pl.ds(g * _LANES, _LANES)] for d in range(_D)]
                for n in range(_N):
                    qvec = bvec * _N + n
                    acc = jnp.zeros((_LANES,), jnp.float32)
                    for d in range(_D):
                        pv = plsc.load_gather(
                            prows, [qvec, jnp.full((_LANES,), d, jnp.int32)])
                        acc = acc + us[d] * pv
                    plsc.store_scatter(
                        outv, [bvec, jnp.full((_LANES,), n, jnp.int32)], acc)
                return carry2

            lax.fori_loop(0, _CH // _LANES, group_body, 0)
            pltpu.sync_copy(outv, out_hbm.at[pl.ds(gbase, _CH)])
            return carry

        lax.fori_loop(0, _NCH, chunk_body, 0)

    return body(sess_flat, prods_flat, uembT, pemb)


def kernel(session, products, user_embedding, product_embedding):
    sess_flat = session.reshape(-1)
    prods_flat = products.reshape(-1)
    return _recsys_call(sess_flat, prods_flat, user_embedding.T,
                        product_embedding)
